# baseline (speedup 1.0000x reference)
import jax
import jax.numpy as jnp
from jax.experimental import pallas as pl
from jax.experimental.pallas import tpu as pltpu

_UNROLL = 32


def _hybrid_kernel(wid_ref, pid_ref, wtab_ref, ptab_ref, out_ref, tile_ref):
    tm, dim = out_ref.shape
    n_chunks = dim // 128
    num_pos = ptab_ref.shape[0]

    pids = pid_ref[...]
    iota = jax.lax.broadcasted_iota(jnp.int32, (tm, num_pos), 1)
    oh = (pids == iota).astype(jnp.float32).astype(jnp.bfloat16)
    out_ref[...] = jnp.dot(oh, ptab_ref[...],
                           preferred_element_type=jnp.float32)

    def group(g, carry):
        base = g * _UNROLL
        for u in range(_UNROLL):
            t = base + u
            wi8 = pl.multiple_of(wid_ref[0, 0, t] * 8, 8)
            tile_ref[pl.ds(t, 1)] = wtab_ref[pl.ds(wi8, 8), :][None]
        return carry

    jax.lax.fori_loop(0, tm // _UNROLL, group, 0)

    for j in range(n_chunks):
        sl = slice(j * 128, (j + 1) * 128)
        out_ref[:, sl] = out_ref[:, sl] + tile_ref[:, j, :]


def _word_only_kernel(wid_ref, wtab_ref, out_ref):
    tm = out_ref.shape[0]

    def chunk(c, carry):
        base = c * _UNROLL
        for u in range(_UNROLL):
            t = base + u
            wi = wid_ref[0, 0, t]
            out_ref[t, 0] = wtab_ref[wi, 0]
        return carry

    jax.lax.fori_loop(0, tm // _UNROLL, chunk, 0)


def _round_up(x: int, m: int) -> int:
    return ((x + m - 1) // m) * m


def _word_only(word_table, flat_w, n, orig_shape, block_tm):
    vocab, dim = word_table.shape
    tm = max(_UNROLL, min(block_tm, _round_up(n, _UNROLL)))
    n_pad = _round_up(n, tm)
    pad = n_pad - n
    n_blocks = n_pad // tm
    w_ids = jnp.pad(flat_w, (0, pad)).reshape(n_blocks, 1, tm)
    wtab3 = word_table.reshape(vocab, 1, dim)
    out = pl.pallas_call(
        _word_only_kernel,
        out_shape=jax.ShapeDtypeStruct((n_pad, 1, dim), word_table.dtype),
        grid=(n_blocks,),
        in_specs=[
            pl.BlockSpec((1, 1, tm), lambda i: (i, 0, 0),
                         memory_space=pltpu.SMEM),
            pl.BlockSpec((vocab, 1, dim), lambda i: (0, 0, 0)),
        ],
        out_specs=pl.BlockSpec((tm, 1, dim), lambda i: (i, 0, 0)),
        compiler_params=pltpu.CompilerParams(
            dimension_semantics=("arbitrary",),
            vmem_limit_bytes=60 * 1024 * 1024,
        ),
    )(w_ids, wtab3)
    return out[:n, 0].reshape(orig_shape + (dim,))


def seq_gnn_node_embedding_fast(word_table, pos_table, input_ids,
                                position_ids=None, *, add_position=True,
                                block_tm=2048):
    vocab, dim = word_table.shape
    orig_shape = input_ids.shape

    flat_w = input_ids.reshape(-1).astype(jnp.int32)
    n = flat_w.shape[0]
    if n == 0:
        return jnp.zeros(orig_shape + (dim,), dtype=word_table.dtype)

    use_pos = add_position and (position_ids is not None)
    if not use_pos or dim % 128 != 0 or dim > 1024:
        return _word_only(word_table, flat_w, n, orig_shape, block_tm)

    max_pos = pos_table.shape[0]
    tm = max(_UNROLL, min(block_tm, _round_up(n, _UNROLL)))
    n_pad = _round_up(n, tm)
    pad = n_pad - n
    n_blocks = n_pad // tm

    w_ids = jnp.pad(flat_w, (0, pad)).reshape(n_blocks, 1, tm)
    flat_p = jnp.minimum(position_ids.reshape(-1).astype(jnp.int32),
                         max_pos - 1)
    p_ids = jnp.pad(flat_p, (0, pad)).reshape(n_pad, 1)

    wtab_w = jnp.pad(word_table, ((0, 0), (0, 1024 - dim))).reshape(
        vocab * 8, 128)
    ptab_bf = pos_table.astype(jnp.bfloat16)

    out = pl.pallas_call(
        _hybrid_kernel,
        out_shape=jax.ShapeDtypeStruct((n_pad, dim), jnp.float32),
        grid=(n_blocks,),
        in_specs=[
            pl.BlockSpec((1, 1, tm), lambda i: (i, 0, 0),
                         memory_space=pltpu.SMEM),
            pl.BlockSpec((tm, 1), lambda i: (i, 0)),
            pl.BlockSpec((vocab * 8, 128), lambda i: (0, 0)),
            pl.BlockSpec((max_pos, dim), lambda i: (0, 0)),
        ],
        out_specs=pl.BlockSpec((tm, dim), lambda i: (i, 0)),
        scratch_shapes=[pltpu.VMEM((tm, 8, 128), jnp.float32)],
        compiler_params=pltpu.CompilerParams(
            dimension_semantics=("arbitrary",),
            vmem_limit_bytes=60 * 1024 * 1024,
        ),
    )(w_ids, p_ids, wtab_w, ptab_bf)

    return out[:n].reshape(orig_shape + (dim,))


def kernel(word_table, pos_table, input_ids, position_ids):
    return seq_gnn_node_embedding_fast(word_table, pos_table, input_ids,
                                       position_ids)


# speedup vs baseline: 1.2967x; 1.2967x over previous
import jax
import jax.numpy as jnp
from jax.experimental import pallas as pl
from jax.experimental.pallas import tpu as pltpu

_UNROLL = 32


def _hybrid_kernel(wid_ref, pid_ref, wtab_ref, ptab_ref, out_ref, tile_ref):
    tm, dim = out_ref.shape
    n_chunks = dim // 128
    num_pos = ptab_ref.shape[0]

    pids = pid_ref[...]
    iota = jax.lax.broadcasted_iota(jnp.int32, (tm, num_pos), 1)
    oh = (pids == iota).astype(jnp.float32).astype(jnp.bfloat16)
    out_ref[...] = jnp.dot(oh, ptab_ref[...],
                           preferred_element_type=jnp.float32)

    S = tm + 8

    def group(g, carry):
        base = g * _UNROLL
        for u in range(_UNROLL):
            t = base + u
            wi8 = pl.multiple_of(wid_ref[0, 0, t] * 8, 8)
            tile_ref[pl.Slice(t, 8, S), :] = wtab_ref[pl.ds(wi8, 8), :]
        return carry

    jax.lax.fori_loop(0, tm // _UNROLL, group, 0)

    for j in range(n_chunks):
        sl = slice(j * 128, (j + 1) * 128)
        out_ref[:, sl] = out_ref[:, sl] + tile_ref[pl.ds(j * S, tm), :]


def _word_only_kernel(wid_ref, wtab_ref, out_ref):
    tm = out_ref.shape[0]

    def chunk(c, carry):
        base = c * _UNROLL
        for u in range(_UNROLL):
            t = base + u
            wi = wid_ref[0, 0, t]
            out_ref[t, 0] = wtab_ref[wi, 0]
        return carry

    jax.lax.fori_loop(0, tm // _UNROLL, chunk, 0)


def _round_up(x: int, m: int) -> int:
    return ((x + m - 1) // m) * m


def _word_only(word_table, flat_w, n, orig_shape, block_tm):
    vocab, dim = word_table.shape
    tm = max(_UNROLL, min(block_tm, _round_up(n, _UNROLL)))
    n_pad = _round_up(n, tm)
    pad = n_pad - n
    n_blocks = n_pad // tm
    w_ids = jnp.pad(flat_w, (0, pad)).reshape(n_blocks, 1, tm)
    wtab3 = word_table.reshape(vocab, 1, dim)
    out = pl.pallas_call(
        _word_only_kernel,
        out_shape=jax.ShapeDtypeStruct((n_pad, 1, dim), word_table.dtype),
        grid=(n_blocks,),
        in_specs=[
            pl.BlockSpec((1, 1, tm), lambda i: (i, 0, 0),
                         memory_space=pltpu.SMEM),
            pl.BlockSpec((vocab, 1, dim), lambda i: (0, 0, 0)),
        ],
        out_specs=pl.BlockSpec((tm, 1, dim), lambda i: (i, 0, 0)),
        compiler_params=pltpu.CompilerParams(
            dimension_semantics=("arbitrary",),
            vmem_limit_bytes=60 * 1024 * 1024,
        ),
    )(w_ids, wtab3)
    return out[:n, 0].reshape(orig_shape + (dim,))


def seq_gnn_node_embedding_fast(word_table, pos_table, input_ids,
                                position_ids=None, *, add_position=True,
                                block_tm=2048):
    vocab, dim = word_table.shape
    orig_shape = input_ids.shape

    flat_w = input_ids.reshape(-1).astype(jnp.int32)
    n = flat_w.shape[0]
    if n == 0:
        return jnp.zeros(orig_shape + (dim,), dtype=word_table.dtype)

    use_pos = add_position and (position_ids is not None)
    if not use_pos or dim % 128 != 0 or dim > 1024:
        return _word_only(word_table, flat_w, n, orig_shape, block_tm)

    max_pos = pos_table.shape[0]
    tm = max(_UNROLL, min(block_tm, _round_up(n, _UNROLL)))
    n_pad = _round_up(n, tm)
    pad = n_pad - n
    n_blocks = n_pad // tm

    w_ids = jnp.pad(flat_w, (0, pad)).reshape(n_blocks, 1, tm)
    flat_p = jnp.minimum(position_ids.reshape(-1).astype(jnp.int32),
                         max_pos - 1)
    p_ids = jnp.pad(flat_p, (0, pad)).reshape(n_pad, 1)

    wtab_w = jnp.pad(word_table, ((0, 0), (0, 1024 - dim))).reshape(
        vocab * 8, 128)
    ptab_bf = pos_table.astype(jnp.bfloat16)

    out = pl.pallas_call(
        _hybrid_kernel,
        out_shape=jax.ShapeDtypeStruct((n_pad, dim), jnp.float32),
        grid=(n_blocks,),
        in_specs=[
            pl.BlockSpec((1, 1, tm), lambda i: (i, 0, 0),
                         memory_space=pltpu.SMEM),
            pl.BlockSpec((tm, 1), lambda i: (i, 0)),
            pl.BlockSpec((vocab * 8, 128), lambda i: (0, 0)),
            pl.BlockSpec((max_pos, dim), lambda i: (0, 0)),
        ],
        out_specs=pl.BlockSpec((tm, dim), lambda i: (i, 0)),
        scratch_shapes=[pltpu.VMEM((8 * (tm + 8), 128), jnp.float32)],
        compiler_params=pltpu.CompilerParams(
            dimension_semantics=("arbitrary",),
            vmem_limit_bytes=60 * 1024 * 1024,
        ),
    )(w_ids, p_ids, wtab_w, ptab_bf)

    return out[:n].reshape(orig_shape + (dim,))


def kernel(word_table, pos_table, input_ids, position_ids):
    return seq_gnn_node_embedding_fast(word_table, pos_table, input_ids,
                                       position_ids)
